# baseline (speedup 1.0000x reference)
import functools

import jax
import jax.numpy as jnp
from jax import lax
from jax.experimental import pallas as pl
from jax.experimental.pallas import tpu as pltpu
from jax.experimental.pallas import tpu_sc as plsc

_NC = 2
_NS = 16
_NW = _NC * _NS

_LANES = 16


def _node_tables_kernel(x_ref, w1_ref, w2_ref, a_ref, b_ref):
    xv = x_ref[...]
    a_ref[...] = jnp.dot(xv, w1_ref[...], preferred_element_type=jnp.float32)
    b_ref[...] = jnp.dot(xv, w2_ref[...], preferred_element_type=jnp.float32)


def _edge_final_kernel(g_ref, ea_ref, w3_ref, bias_ref, out_ref):
    out_ref[...] = (
        g_ref[...]
        + jnp.dot(ea_ref[...], w3_ref[...], preferred_element_type=jnp.float32)
        + bias_ref[...]
    )


def _edge_final_kernel2(g_ref, ea_ref, w3_ref, bias_ref, prev_ref, out_ref):
    del prev_ref
    out_ref[...] = (
        g_ref[...]
        + jnp.dot(ea_ref[...], w3_ref[...], preferred_element_type=jnp.float32)
        + bias_ref[...]
    )


def _make_sc_gather_add(EH, D, CH):
    per_w = EH // _NW
    nchunk = per_w // CH
    jperrow = D // _LANES
    mesh = plsc.VectorSubcoreMesh(core_axis_name="c", subcore_axis_name="s")

    @functools.partial(
        pl.kernel,
        out_type=jax.ShapeDtypeStruct((EH, D), jnp.float32),
        mesh=mesh,
        scratch_types=[
            pltpu.VMEM((CH,), jnp.int32),
            pltpu.VMEM((CH,), jnp.int32),
            pltpu.VMEM((CH,), jnp.int32),
            pltpu.VMEM((CH,), jnp.int32),
            pltpu.VMEM((CH, D), jnp.float32),
            pltpu.VMEM((CH, D), jnp.float32),
            pltpu.VMEM((CH, D), jnp.float32),
            pltpu.VMEM((CH, D), jnp.float32),
        ] + [pltpu.SemaphoreType.DMA] * 10,
    )
    def sc_edge_kernel(a_hbm, b_hbm, row_hbm, col_hbm, out_hbm,
                       row0, col0, row1, col1, bufa0, bufb0, bufa1, bufb1,
                       semr0, semc0, semr1, semc1,
                       sema0, semb0, sema1, semb1, semst0, semst1):
        rowv = (row0, row1)
        colv = (col0, col1)
        bufa = (bufa0, bufa1)
        bufb = (bufb0, bufb1)
        semr = (semr0, semr1)
        semc = (semc0, semc1)
        sema = (sema0, sema1)
        semb = (semb0, semb1)
        semst = (semst0, semst1)

        wid = lax.axis_index("s") * _NC + lax.axis_index("c")
        base_w = wid * per_w

        def issue_idx(i):
            p = i % 2
            base = base_w + i * CH
            return (
                pltpu.async_copy(row_hbm.at[pl.ds(base, CH)], rowv[p], semr[p]),
                pltpu.async_copy(col_hbm.at[pl.ds(base, CH)], colv[p], semc[p]),
            )

        def issue_gather(i):
            p = i % 2
            return (
                pltpu.async_copy(a_hbm.at[rowv[p]], bufa[p], sema[p]),
                pltpu.async_copy(b_hbm.at[colv[p]], bufb[p], semb[p]),
            )

        idxd = [None] * (nchunk + 1)
        gd = [None] * (nchunk + 1)
        std = [None] * (nchunk + 1)

        idxd[0] = issue_idx(0)
        if nchunk > 1:
            idxd[1] = issue_idx(1)
        idxd[0][0].wait()
        idxd[0][1].wait()
        gd[0] = issue_gather(0)

        for i in range(nchunk):
            p = i % 2
            if i + 1 < nchunk:
                if i >= 1:
                    std[i - 1].wait()
                idxd[i + 1][0].wait()
                idxd[i + 1][1].wait()
                gd[i + 1] = issue_gather(i + 1)
            gd[i][0].wait()
            gd[i][1].wait()
            if i + 2 < nchunk:
                idxd[i + 2] = issue_idx(i + 2)

            ba, bb = bufa[p], bufb[p]

            def add_body(e, c2):
                for j in range(jperrow):
                    sl = pl.ds(j * _LANES, _LANES)
                    ba[e, sl] = ba[e, sl] + bb[e, sl]
                return c2

            lax.fori_loop(0, CH, add_body, 0, unroll=False)
            std[i] = pltpu.async_copy(
                ba, out_hbm.at[pl.ds(base_w + i * CH, CH)], semst[p])

        if nchunk >= 2:
            std[nchunk - 2].wait()
        std[nchunk - 1].wait()

    return sc_edge_kernel


def kernel(x, edge_index, edge_attr, edge_type, W, b):
    del edge_type
    N, D = x.shape
    E, DE = edge_attr.shape
    DOUT = W.shape[1]

    W1 = lax.slice(W, (0, 0), (D, DOUT))
    W2 = lax.slice(W, (D, 0), (2 * D, DOUT))
    W3 = lax.slice(W, (2 * D, 0), (2 * D + DE, DOUT))
    b2 = b.reshape(1, DOUT)
    row = edge_index[0]
    col = edge_index[1]

    A, B = pl.pallas_call(
        _node_tables_kernel,
        out_shape=[
            jax.ShapeDtypeStruct((N, DOUT), jnp.float32),
            jax.ShapeDtypeStruct((N, DOUT), jnp.float32),
        ],
    )(x, W1, W2)

    splits = [32000, 96000, 96000, 64000, 32000]
    assert sum(splits) == E
    BE = 3200
    makers = {}
    gs = []
    off = 0
    for sz in splits:
        if sz not in makers:
            makers[sz] = _make_sc_gather_add(sz, DOUT, CH=200)
        r = lax.slice(row, (off,), (off + sz,))
        c = lax.slice(col, (off,), (off + sz,))
        gs.append(makers[sz](A, B, r, c))
        off += sz

    out = None
    off_b = 0
    for k, sz in enumerate(splits):
        nb = sz // BE
        ob = off_b
        common = dict(
            grid=(nb,),
            out_specs=pl.BlockSpec((BE, DOUT), lambda i, ob=ob: (i + ob, 0)),
            out_shape=jax.ShapeDtypeStruct((E, DOUT), jnp.float32),
        )
        in_specs = [
            pl.BlockSpec((BE, DOUT), lambda i: (i, 0)),
            pl.BlockSpec((BE, DE), lambda i, ob=ob: (i + ob, 0)),
            pl.BlockSpec((DE, DOUT), lambda i: (0, 0)),
            pl.BlockSpec((1, DOUT), lambda i: (0, 0)),
        ]
        if k == 0:
            out = pl.pallas_call(
                _edge_final_kernel, in_specs=in_specs, **common,
            )(gs[k], edge_attr, W3, b2)
        else:
            out = pl.pallas_call(
                _edge_final_kernel2,
                in_specs=in_specs + [pl.BlockSpec(memory_space=pl.ANY)],
                input_output_aliases={4: 0}, **common,
            )(gs[k], edge_attr, W3, b2, out)
        off_b += nb
    return out


# speedup vs baseline: 1.0008x; 1.0008x over previous
import functools

import jax
import jax.numpy as jnp
from jax import lax
from jax.experimental import pallas as pl
from jax.experimental.pallas import tpu as pltpu
from jax.experimental.pallas import tpu_sc as plsc

_NC = 2
_NS = 16
_NW = _NC * _NS

_LANES = 16


def _node_tables_kernel(x_ref, w1_ref, w2_ref, a_ref, b_ref):
    xv = x_ref[...]
    a_ref[...] = jnp.dot(xv, w1_ref[...], preferred_element_type=jnp.float32)
    b_ref[...] = jnp.dot(xv, w2_ref[...], preferred_element_type=jnp.float32)


def _edge_final_kernel(g_ref, ea_ref, w3_ref, bias_ref, out_ref):
    out_ref[...] = (
        g_ref[...]
        + jnp.dot(ea_ref[...], w3_ref[...], preferred_element_type=jnp.float32)
        + bias_ref[...]
    )


def _edge_final_kernel2(g_ref, ea_ref, w3_ref, bias_ref, prev_ref, out_ref):
    del prev_ref
    out_ref[...] = (
        g_ref[...]
        + jnp.dot(ea_ref[...], w3_ref[...], preferred_element_type=jnp.float32)
        + bias_ref[...]
    )


def _make_sc_gather_add(EH, D, CH):
    per_w = EH // _NW
    nchunk = per_w // CH
    jperrow = D // _LANES
    mesh = plsc.VectorSubcoreMesh(core_axis_name="c", subcore_axis_name="s")

    @functools.partial(
        pl.kernel,
        out_type=jax.ShapeDtypeStruct((EH, D), jnp.float32),
        mesh=mesh,
        scratch_types=[
            pltpu.VMEM((CH,), jnp.int32),
            pltpu.VMEM((CH,), jnp.int32),
            pltpu.VMEM((CH,), jnp.int32),
            pltpu.VMEM((CH,), jnp.int32),
            pltpu.VMEM((CH, D), jnp.float32),
            pltpu.VMEM((CH, D), jnp.float32),
            pltpu.VMEM((CH, D), jnp.float32),
            pltpu.VMEM((CH, D), jnp.float32),
        ] + [pltpu.SemaphoreType.DMA] * 10,
    )
    def sc_edge_kernel(a_hbm, b_hbm, row_hbm, col_hbm, out_hbm,
                       row0, col0, row1, col1, bufa0, bufb0, bufa1, bufb1,
                       semr0, semc0, semr1, semc1,
                       sema0, semb0, sema1, semb1, semst0, semst1):
        rowv = (row0, row1)
        colv = (col0, col1)
        bufa = (bufa0, bufa1)
        bufb = (bufb0, bufb1)
        semr = (semr0, semr1)
        semc = (semc0, semc1)
        sema = (sema0, sema1)
        semb = (semb0, semb1)
        semst = (semst0, semst1)

        wid = lax.axis_index("s") * _NC + lax.axis_index("c")
        base_w = wid * per_w

        def issue_idx(i):
            p = i % 2
            base = base_w + i * CH
            return (
                pltpu.async_copy(row_hbm.at[pl.ds(base, CH)], rowv[p], semr[p]),
                pltpu.async_copy(col_hbm.at[pl.ds(base, CH)], colv[p], semc[p]),
            )

        def issue_gather(i):
            p = i % 2
            return (
                pltpu.async_copy(a_hbm.at[rowv[p]], bufa[p], sema[p]),
                pltpu.async_copy(b_hbm.at[colv[p]], bufb[p], semb[p]),
            )

        idxd = [None] * (nchunk + 1)
        gd = [None] * (nchunk + 1)
        std = [None] * (nchunk + 1)

        idxd[0] = issue_idx(0)
        if nchunk > 1:
            idxd[1] = issue_idx(1)
        idxd[0][0].wait()
        idxd[0][1].wait()
        gd[0] = issue_gather(0)

        for i in range(nchunk):
            p = i % 2
            if i + 1 < nchunk:
                if i >= 1:
                    std[i - 1].wait()
                idxd[i + 1][0].wait()
                idxd[i + 1][1].wait()
                gd[i + 1] = issue_gather(i + 1)
            gd[i][0].wait()
            gd[i][1].wait()
            if i + 2 < nchunk:
                idxd[i + 2] = issue_idx(i + 2)

            ba, bb = bufa[p], bufb[p]

            def add_body(e, c2):
                for j in range(jperrow):
                    sl = pl.ds(j * _LANES, _LANES)
                    ba[e, sl] = ba[e, sl] + bb[e, sl]
                return c2

            lax.fori_loop(0, CH, add_body, 0, unroll=False)
            std[i] = pltpu.async_copy(
                ba, out_hbm.at[pl.ds(base_w + i * CH, CH)], semst[p])

        if nchunk >= 2:
            std[nchunk - 2].wait()
        std[nchunk - 1].wait()

    return sc_edge_kernel


def kernel(x, edge_index, edge_attr, edge_type, W, b):
    del edge_type
    N, D = x.shape
    E, DE = edge_attr.shape
    DOUT = W.shape[1]

    W1 = lax.slice(W, (0, 0), (D, DOUT))
    W2 = lax.slice(W, (D, 0), (2 * D, DOUT))
    W3 = lax.slice(W, (2 * D, 0), (2 * D + DE, DOUT))
    b2 = b.reshape(1, DOUT)
    row = edge_index[0]
    col = edge_index[1]

    A, B = pl.pallas_call(
        _node_tables_kernel,
        out_shape=[
            jax.ShapeDtypeStruct((N, DOUT), jnp.float32),
            jax.ShapeDtypeStruct((N, DOUT), jnp.float32),
        ],
    )(x, W1, W2)

    splits = [32000, 96000, 96000, 64000, 32000]
    assert sum(splits) == E
    BE = 3200
    makers = {}
    offs = []
    off = 0
    for sz in splits:
        if sz not in makers:
            makers[sz] = _make_sc_gather_add(sz, DOUT, CH=200)
        offs.append(off)
        off += sz

    def run_sc(k):
        sz = splits[k]
        o = offs[k]
        r = lax.slice(row, (o,), (o + sz,))
        c = lax.slice(col, (o,), (o + sz,))
        return makers[sz](A, B, r, c)

    gs = [None] * len(splits)
    gs[0] = run_sc(0)

    out = None
    off_b = 0
    for k, sz in enumerate(splits):
        if k + 1 < len(splits):
            gs[k + 1] = run_sc(k + 1)
        nb = sz // BE
        ob = off_b
        common = dict(
            grid=(nb,),
            out_specs=pl.BlockSpec((BE, DOUT), lambda i, ob=ob: (i + ob, 0)),
            out_shape=jax.ShapeDtypeStruct((E, DOUT), jnp.float32),
        )
        in_specs = [
            pl.BlockSpec((BE, DOUT), lambda i: (i, 0)),
            pl.BlockSpec((BE, DE), lambda i, ob=ob: (i + ob, 0)),
            pl.BlockSpec((DE, DOUT), lambda i: (0, 0)),
            pl.BlockSpec((1, DOUT), lambda i: (0, 0)),
        ]
        if k == 0:
            out = pl.pallas_call(
                _edge_final_kernel, in_specs=in_specs, **common,
            )(gs[k], edge_attr, W3, b2)
        else:
            out = pl.pallas_call(
                _edge_final_kernel2,
                in_specs=in_specs + [pl.BlockSpec(memory_space=pl.ANY)],
                input_output_aliases={4: 0}, **common,
            )(gs[k], edge_attr, W3, b2, out)
        off_b += nb
    return out
